# Initial kernel scaffold; baseline (speedup 1.0000x reference)
import jax
import jax.numpy as jnp
from jax.experimental import pallas as pl


def kernel(inputs, unpool_mat, output_shape):
    raise NotImplementedError("write your pallas kernel here")


# speedup vs baseline: 3.6027x; 3.6027x over previous
import functools

import jax
import jax.numpy as jnp
from jax import lax
from jax.experimental import pallas as pl
from jax.experimental.pallas import tpu as pltpu
from jax.experimental.pallas import tpu_sc as plsc

_NC = 2
_NS = 16
_NW = _NC * _NS

_CH = 65536
_WIN = 4096
_GRP = _WIN // 16
_SENT = 2**30


def _sc_scatter(ks_pad, vs_pad, bnd, M, n_full, part):
    R = M // _NW
    n_bnd = bnd.shape[0]
    mesh = plsc.VectorSubcoreMesh(core_axis_name="c", subcore_axis_name="s")

    @functools.partial(
        pl.kernel,
        out_type=jax.ShapeDtypeStruct((M,), jnp.float32),
        mesh=mesh,
        scratch_types=[
            pltpu.VMEM((_CH,), jnp.float32),
            pltpu.VMEM((_WIN + 16,), jnp.int32),
            pltpu.VMEM((_WIN,), jnp.float32),
            pltpu.VMEM((n_bnd,), jnp.int32),
        ],
        compiler_params=pltpu.CompilerParams(needs_layout_passes=False),
    )
    def body(ks_hbm, vs_hbm, bnd_hbm, out_hbm, chunk_v, kwin_v, vwin_v, bnd_v):
        cid = lax.axis_index("c")
        sid = lax.axis_index("s")
        w = sid * _NC + cid
        pltpu.sync_copy(bnd_hbm, bnd_v)
        zvec = jnp.zeros((16,), jnp.float32)

        def do_chunk(kglob, o0, nwords):
            def zb(i, _):
                chunk_v[pl.ds(i * 16, 16)] = zvec
                return 0

            lax.fori_loop(0, nwords // 16, zb, 0)
            bvec = bnd_v[pl.ds(kglob, 16)]
            lo = bvec[0]
            hi = bvec[1]
            lo_al = (lo // 16) * 16
            nwin = (hi - lo_al + (_WIN - 1)) // _WIN

            def wbody(t, _):
                base = lo_al + t * _WIN
                pltpu.sync_copy(ks_hbm.at[pl.ds(base, _WIN + 16)], kwin_v)
                pltpu.sync_copy(vs_hbm.at[pl.ds(base, _WIN)], vwin_v)

                def gbody(g, _2):
                    k0 = kwin_v[pl.ds(g * 16, 16)]
                    k1 = kwin_v[pl.ds(g * 16 + 1, 16)]
                    v = vwin_v[pl.ds(g * 16, 16)]
                    local = k0 - o0
                    m = (k0 != k1) & (local >= 0) & (local < nwords)
                    plsc.store_scatter(chunk_v, [local], v, mask=m)
                    return 0

                lax.fori_loop(0, _GRP, gbody, 0)
                return 0

            lax.fori_loop(0, nwin, wbody, 0)
            pltpu.sync_copy(chunk_v.at[pl.ds(0, nwords)],
                            out_hbm.at[pl.ds(o0, nwords)])

        def full_chunk(ci, _):
            do_chunk(w * (n_full + 1) + ci, w * R + ci * _CH, _CH)
            return 0

        lax.fori_loop(0, n_full, full_chunk, 0)
        do_chunk(w * (n_full + 1) + n_full, w * R + n_full * _CH, part)

    return body(ks_pad, vs_pad, bnd)


def kernel(inputs, unpool_mat, output_shape):
    B, Hp, Wp, C = inputs.shape
    Ho, Wo = 2 * Hp, 2 * Wp
    M = B * Ho * Wo * C
    N = B * Hp * Wp * C

    vals = jnp.reshape(inputs, (-1,))
    idx = jnp.reshape(unpool_mat, (-1,))
    if idx.dtype != jnp.int32:
        idx = idx.astype(jnp.int32)
    traced_prod = jnp.prod(jnp.asarray(output_shape, dtype=idx.dtype))
    keys = idx + (traced_prod - M)
    keys = jnp.where(keys < 0, keys + M, keys)

    ks, vs = lax.sort((keys, vals), dimension=0, num_keys=1, is_stable=False)

    ks_pad = jnp.concatenate(
        [ks, jnp.full((_WIN + 16,), _SENT, jnp.int32)])
    vs_pad = jnp.concatenate([vs, jnp.zeros((_WIN,), jnp.float32)])

    R = M // _NW
    n_full = (R // _CH) if (R % _CH) else (R // _CH - 1)
    part = R - n_full * _CH
    npc = n_full + 1
    starts = []
    for w in range(_NW):
        for c in range(npc):
            starts.append(w * R + c * _CH)
    starts.append(M)
    q = jnp.asarray(starts, dtype=jnp.int32)
    bnd = jnp.searchsorted(ks, q).astype(jnp.int32)
    n_bnd = ((_NW * npc + 1 + 15) // 16) * 16
    bnd = jnp.concatenate(
        [bnd, jnp.full((n_bnd - bnd.shape[0],), N, jnp.int32)])

    out = _sc_scatter(ks_pad, vs_pad, bnd, M, n_full, part)
    return jnp.reshape(out, (B, Ho, Wo, C))


# speedup vs baseline: 3.6988x; 1.0267x over previous
import functools

import jax
import jax.numpy as jnp
from jax import lax
from jax.experimental import pallas as pl
from jax.experimental.pallas import tpu as pltpu
from jax.experimental.pallas import tpu_sc as plsc

_NC = 2
_NS = 16
_NW = _NC * _NS

_CH = 32768
_WIN = 4096
_GRP = _WIN // 16
_SENT = 2**30


def _sc_scatter(ks_pad, vs_pad, bnd, M, n_full, part):
    R = M // _NW
    n_bnd = bnd.shape[0]
    npc = n_full + 1
    n_half = n_full // 2
    mesh = plsc.VectorSubcoreMesh(core_axis_name="c", subcore_axis_name="s")

    @functools.partial(
        pl.kernel,
        out_type=jax.ShapeDtypeStruct((M,), jnp.float32),
        mesh=mesh,
        scratch_types=[
            pltpu.VMEM((_CH,), jnp.float32),
            pltpu.VMEM((_CH,), jnp.float32),
            pltpu.VMEM((_WIN + 16,), jnp.int32),
            pltpu.VMEM((_WIN + 16,), jnp.int32),
            pltpu.VMEM((_WIN,), jnp.float32),
            pltpu.VMEM((_WIN,), jnp.float32),
            pltpu.VMEM((n_bnd,), jnp.int32),
            pltpu.SemaphoreType.DMA,
            pltpu.SemaphoreType.DMA,
            pltpu.SemaphoreType.DMA,
            pltpu.SemaphoreType.DMA,
            pltpu.SemaphoreType.DMA,
            pltpu.SemaphoreType.DMA,
        ],
        compiler_params=pltpu.CompilerParams(needs_layout_passes=False),
    )
    def body(ks_hbm, vs_hbm, bnd_hbm, out_hbm,
             chA, chB, kwA, kwB, vwA, vwB, bnd_v,
             ksA, ksB, vsA, vsB, osA, osB):
        cid = lax.axis_index("c")
        sid = lax.axis_index("s")
        w = sid * _NC + cid
        pltpu.sync_copy(bnd_hbm, bnd_v)
        zvec = jnp.zeros((16,), jnp.float32)
        iota = lax.iota(jnp.int32, 16)

        def issue_win(t_base, kw, vw, ksem, vsem):
            pltpu.async_copy(ks_hbm.at[pl.ds(t_base, _WIN + 16)], kw, ksem)
            pltpu.async_copy(vs_hbm.at[pl.ds(t_base, _WIN)], vw, vsem)

        def wait_win(kw, vw, ksem, vsem):
            pltpu.make_async_copy(ks_hbm.at[pl.ds(0, _WIN + 16)], kw, ksem).wait()
            pltpu.make_async_copy(vs_hbm.at[pl.ds(0, _WIN)], vw, vsem).wait()

        def process_win(chunk_v, kw, vw, o0, nwords, lo16, hi, t):
            base = lo16 + t * _WIN

            @plsc.parallel_loop(0, _GRP, unroll=4)
            def _(g):
                k0 = kw[pl.ds(g * 16, 16)]
                k1 = kw[pl.ds(g * 16 + 1, 16)]
                v = vw[pl.ds(g * 16, 16)]
                local = k0 - o0
                m = ((k0 != k1) & (local >= 0) & (local < nwords)
                     & ((base + g * 16 + iota) < hi))
                plsc.store_scatter(chunk_v, [local], v, mask=m)

        def do_chunk(kglob, o0, nwords, chunk_v, kws, vws, ksems, vsems):
            @plsc.parallel_loop(0, nwords // 16, unroll=8)
            def _(i):
                chunk_v[pl.ds(i * 16, 16)] = zvec

            bvec = bnd_v[pl.ds(kglob, 16)]
            lo = bvec[0]
            hi = bvec[1]
            lo16 = (lo // 16) * 16
            nwin = (hi - lo16 + (_WIN - 1)) // _WIN

            @pl.when(nwin > 0)
            def _():
                issue_win(lo16, kws[0], vws[0], ksems[0], vsems[0])

                def wpair(t, _c):
                    wait_win(kws[0], vws[0], ksems[0], vsems[0])

                    @pl.when(2 * t + 1 < nwin)
                    def _():
                        issue_win(lo16 + (2 * t + 1) * _WIN,
                                  kws[1], vws[1], ksems[1], vsems[1])

                    process_win(chunk_v, kws[0], vws[0], o0, nwords, lo16,
                                hi, 2 * t)

                    @pl.when(2 * t + 2 < nwin)
                    def _():
                        issue_win(lo16 + (2 * t + 2) * _WIN,
                                  kws[0], vws[0], ksems[0], vsems[0])

                    @pl.when(2 * t + 1 < nwin)
                    def _():
                        wait_win(kws[1], vws[1], ksems[1], vsems[1])
                        process_win(chunk_v, kws[1], vws[1], o0, nwords,
                                    lo16, hi, 2 * t + 1)

                    return 0

                lax.fori_loop(0, (nwin + 1) // 2, wpair, 0)

        def out_start(chunk_v, o0, nwords, osem):
            pltpu.async_copy(chunk_v.at[pl.ds(0, nwords)],
                             out_hbm.at[pl.ds(o0, nwords)], osem)

        def out_wait(chunk_v, nwords, osem):
            pltpu.make_async_copy(chunk_v.at[pl.ds(0, nwords)],
                                  out_hbm.at[pl.ds(0, nwords)], osem).wait()

        def cpair(i, _c):
            ci = 2 * i

            @pl.when(i > 0)
            def _():
                out_wait(chA, _CH, osA)

            do_chunk(w * npc + ci, w * R + ci * _CH, _CH, chA,
                     (kwA, kwB), (vwA, vwB), (ksA, ksB), (vsA, vsB))
            out_start(chA, w * R + ci * _CH, _CH, osA)

            @pl.when(i > 0)
            def _():
                out_wait(chB, _CH, osB)

            do_chunk(w * npc + ci + 1, w * R + (ci + 1) * _CH, _CH, chB,
                     (kwA, kwB), (vwA, vwB), (ksA, ksB), (vsA, vsB))
            out_start(chB, w * R + (ci + 1) * _CH, _CH, osB)
            return 0

        lax.fori_loop(0, n_half, cpair, 0)
        out_wait(chA, _CH, osA)
        do_chunk(w * npc + n_full, w * R + n_full * _CH, part, chA,
                 (kwA, kwB), (vwA, vwB), (ksA, ksB), (vsA, vsB))
        out_start(chA, w * R + n_full * _CH, part, osA)
        out_wait(chA, part, osA)
        out_wait(chB, _CH, osB)

    return body(ks_pad, vs_pad, bnd)


def kernel(inputs, unpool_mat, output_shape):
    B, Hp, Wp, C = inputs.shape
    Ho, Wo = 2 * Hp, 2 * Wp
    M = B * Ho * Wo * C
    N = B * Hp * Wp * C

    vals = jnp.reshape(inputs, (-1,))
    idx = jnp.reshape(unpool_mat, (-1,))
    if idx.dtype != jnp.int32:
        idx = idx.astype(jnp.int32)

    ks, vs = lax.sort((idx, vals), dimension=0, num_keys=1, is_stable=False)

    ks_pad = jnp.concatenate(
        [ks, jnp.full((_WIN + 16,), _SENT, jnp.int32)])
    vs_pad = jnp.concatenate([vs, jnp.zeros((_WIN,), jnp.float32)])

    R = M // _NW
    n_full = (R // _CH) if (R % _CH) else (R // _CH - 1)
    if n_full % 2:
        n_full -= 1
    part = R - n_full * _CH
    assert part <= _CH
    npc = n_full + 1
    starts = []
    for w in range(_NW):
        for c in range(npc):
            starts.append(w * R + c * _CH)
    starts.append(M)
    q = jnp.asarray(starts, dtype=jnp.int32)
    bnd = jnp.searchsorted(ks, q).astype(jnp.int32)
    n_bnd = ((_NW * npc + 1 + 15) // 16) * 16
    bnd = jnp.concatenate(
        [bnd, jnp.full((n_bnd - bnd.shape[0],), N, jnp.int32)])

    out = _sc_scatter(ks_pad, vs_pad, bnd, M, n_full, part)
    return jnp.reshape(out, (B, Ho, Wo, C))


# speedup vs baseline: 3.7129x; 1.0038x over previous
import functools

import jax
import jax.numpy as jnp
from jax import lax
from jax.experimental import pallas as pl
from jax.experimental.pallas import tpu as pltpu
from jax.experimental.pallas import tpu_sc as plsc

_NC = 2
_NS = 16
_NW = _NC * _NS

_CH = 32768
_WIN = 4096
_GRP = _WIN // 16
_SENT = 2**30


def _sc_scatter(ks, vs, bnd, M, n_full, part):
    R = M // _NW
    N = ks.shape[0]
    clamp = N - _WIN - 16
    n_bnd = bnd.shape[0]
    npc = n_full + 1
    n_half = n_full // 2
    mesh = plsc.VectorSubcoreMesh(core_axis_name="c", subcore_axis_name="s")

    @functools.partial(
        pl.kernel,
        out_type=jax.ShapeDtypeStruct((M,), jnp.float32),
        mesh=mesh,
        scratch_types=[
            pltpu.VMEM((_CH,), jnp.float32),
            pltpu.VMEM((_CH,), jnp.float32),
            pltpu.VMEM((_WIN + 48,), jnp.int32),
            pltpu.VMEM((_WIN + 48,), jnp.int32),
            pltpu.VMEM((_WIN + 32,), jnp.float32),
            pltpu.VMEM((_WIN + 32,), jnp.float32),
            pltpu.VMEM((n_bnd,), jnp.int32),
            pltpu.SemaphoreType.DMA,
            pltpu.SemaphoreType.DMA,
            pltpu.SemaphoreType.DMA,
            pltpu.SemaphoreType.DMA,
            pltpu.SemaphoreType.DMA,
            pltpu.SemaphoreType.DMA,
        ],
        compiler_params=pltpu.CompilerParams(needs_layout_passes=False),
    )
    def body(ks_hbm, vs_hbm, bnd_hbm, out_hbm,
             chA, chB, kwA, kwB, vwA, vwB, bnd_v,
             ksA, ksB, vsA, vsB, osA, osB):
        cid = lax.axis_index("c")
        sid = lax.axis_index("s")
        w = sid * _NC + cid
        pltpu.sync_copy(bnd_hbm, bnd_v)
        zvec = jnp.zeros((16,), jnp.float32)
        iota = lax.iota(jnp.int32, 16)

        def issue_win(t_base, kw, vw, ksem, vsem):
            base_c = jnp.minimum(t_base, clamp)
            pltpu.async_copy(ks_hbm.at[pl.ds(base_c, _WIN + 16)],
                             kw.at[pl.ds(0, _WIN + 16)], ksem)
            pltpu.async_copy(vs_hbm.at[pl.ds(base_c, _WIN + 16)],
                             vw.at[pl.ds(0, _WIN + 16)], vsem)

        def wait_win(kw, vw, ksem, vsem):
            pltpu.make_async_copy(ks_hbm.at[pl.ds(0, _WIN + 16)],
                                  kw.at[pl.ds(0, _WIN + 16)], ksem).wait()
            pltpu.make_async_copy(vs_hbm.at[pl.ds(0, _WIN + 16)],
                                  vw.at[pl.ds(0, _WIN + 16)], vsem).wait()

        def process_win(chunk_v, kw, vw, o0, nwords, lo16, hi, t):
            base = lo16 + t * _WIN
            shift = base - jnp.minimum(base, clamp)

            @plsc.parallel_loop(0, _GRP, unroll=4)
            def _(g):
                st = jnp.minimum(g * 16 + shift, _WIN + 16)
                k0 = kw[pl.ds(st, 16)]
                k1 = kw[pl.ds(st + 1, 16)]
                v = vw[pl.ds(st, 16)]
                local = k0 - o0
                pos = base + g * 16 + iota
                m = (((k0 != k1) | (pos == (N - 1)))
                     & (local >= 0) & (local < nwords) & (pos < hi))
                plsc.store_scatter(chunk_v, [local], v, mask=m)

        def do_chunk(kglob, o0, nwords, chunk_v, kws, vws, ksems, vsems):
            @plsc.parallel_loop(0, nwords // 16, unroll=8)
            def _(i):
                chunk_v[pl.ds(i * 16, 16)] = zvec

            bvec = bnd_v[pl.ds(kglob, 16)]
            lo = bvec[0]
            hi = bvec[1]
            lo16 = (lo // 16) * 16
            nwin = (hi - lo16 + (_WIN - 1)) // _WIN

            @pl.when(nwin > 0)
            def _():
                issue_win(lo16, kws[0], vws[0], ksems[0], vsems[0])

                def wpair(t, _c):
                    wait_win(kws[0], vws[0], ksems[0], vsems[0])

                    @pl.when(2 * t + 1 < nwin)
                    def _():
                        issue_win(lo16 + (2 * t + 1) * _WIN,
                                  kws[1], vws[1], ksems[1], vsems[1])

                    process_win(chunk_v, kws[0], vws[0], o0, nwords, lo16,
                                hi, 2 * t)

                    @pl.when(2 * t + 2 < nwin)
                    def _():
                        issue_win(lo16 + (2 * t + 2) * _WIN,
                                  kws[0], vws[0], ksems[0], vsems[0])

                    @pl.when(2 * t + 1 < nwin)
                    def _():
                        wait_win(kws[1], vws[1], ksems[1], vsems[1])
                        process_win(chunk_v, kws[1], vws[1], o0, nwords,
                                    lo16, hi, 2 * t + 1)

                    return 0

                lax.fori_loop(0, (nwin + 1) // 2, wpair, 0)

        def out_start(chunk_v, o0, nwords, osem):
            pltpu.async_copy(chunk_v.at[pl.ds(0, nwords)],
                             out_hbm.at[pl.ds(o0, nwords)], osem)

        def out_wait(chunk_v, nwords, osem):
            pltpu.make_async_copy(chunk_v.at[pl.ds(0, nwords)],
                                  out_hbm.at[pl.ds(0, nwords)], osem).wait()

        def cpair(i, _c):
            ci = 2 * i

            @pl.when(i > 0)
            def _():
                out_wait(chA, _CH, osA)

            do_chunk(w * npc + ci, w * R + ci * _CH, _CH, chA,
                     (kwA, kwB), (vwA, vwB), (ksA, ksB), (vsA, vsB))
            out_start(chA, w * R + ci * _CH, _CH, osA)

            @pl.when(i > 0)
            def _():
                out_wait(chB, _CH, osB)

            do_chunk(w * npc + ci + 1, w * R + (ci + 1) * _CH, _CH, chB,
                     (kwA, kwB), (vwA, vwB), (ksA, ksB), (vsA, vsB))
            out_start(chB, w * R + (ci + 1) * _CH, _CH, osB)
            return 0

        lax.fori_loop(0, n_half, cpair, 0)
        out_wait(chA, _CH, osA)
        do_chunk(w * npc + n_full, w * R + n_full * _CH, part, chA,
                 (kwA, kwB), (vwA, vwB), (ksA, ksB), (vsA, vsB))
        out_start(chA, w * R + n_full * _CH, part, osA)
        out_wait(chA, part, osA)
        out_wait(chB, _CH, osB)

    return body(ks, vs, bnd)


def kernel(inputs, unpool_mat, output_shape):
    B, Hp, Wp, C = inputs.shape
    Ho, Wo = 2 * Hp, 2 * Wp
    M = B * Ho * Wo * C
    N = B * Hp * Wp * C

    vals = jnp.reshape(inputs, (-1,))
    idx = jnp.reshape(unpool_mat, (-1,))
    if idx.dtype != jnp.int32:
        idx = idx.astype(jnp.int32)

    ks, vs = lax.sort((idx, vals), dimension=0, num_keys=1, is_stable=False)

    R = M // _NW
    n_full = (R // _CH) if (R % _CH) else (R // _CH - 1)
    if n_full % 2:
        n_full -= 1
    part = R - n_full * _CH
    assert part <= _CH
    npc = n_full + 1
    starts = []
    for w in range(_NW):
        for c in range(npc):
            starts.append(w * R + c * _CH)
    starts.append(M)
    q = jnp.asarray(starts, dtype=jnp.int32)
    bnd = jnp.searchsorted(ks, q).astype(jnp.int32)
    n_bnd = ((_NW * npc + 1 + 15) // 16) * 16
    bnd = jnp.concatenate(
        [bnd, jnp.full((n_bnd - bnd.shape[0],), N, jnp.int32)])

    out = _sc_scatter(ks, vs, bnd, M, n_full, part)
    return jnp.reshape(out, (B, Ho, Wo, C))


# speedup vs baseline: 3.7130x; 1.0000x over previous
import functools

import jax
import jax.numpy as jnp
from jax import lax
from jax.experimental import pallas as pl
from jax.experimental.pallas import tpu as pltpu
from jax.experimental.pallas import tpu_sc as plsc

_NC = 2
_NS = 16
_NW = _NC * _NS

_CH = 32768
_WIN = 4096
_GRP = _WIN // 16


def _sc_scatter(ks, vs, bnd, M, n_full, part):
    R = M // _NW
    N = ks.shape[0]
    clamp = N - _WIN - 16
    n_bnd = bnd.shape[0]
    npc = n_full + 1
    n_half = n_full // 2
    mesh = plsc.VectorSubcoreMesh(core_axis_name="c", subcore_axis_name="s")

    @functools.partial(
        pl.kernel,
        out_type=jax.ShapeDtypeStruct((M,), jnp.float32),
        mesh=mesh,
        scratch_types=[
            pltpu.VMEM((_CH,), jnp.float32),
            pltpu.VMEM((_CH,), jnp.float32),
            pltpu.VMEM((_WIN + 48,), jnp.int32),
            pltpu.VMEM((_WIN + 48,), jnp.int32),
            pltpu.VMEM((_WIN + 32,), jnp.float32),
            pltpu.VMEM((_WIN + 32,), jnp.float32),
            pltpu.VMEM((n_bnd,), jnp.int32),
            pltpu.SemaphoreType.DMA,
            pltpu.SemaphoreType.DMA,
            pltpu.SemaphoreType.DMA,
            pltpu.SemaphoreType.DMA,
            pltpu.SemaphoreType.DMA,
            pltpu.SemaphoreType.DMA,
        ],
        compiler_params=pltpu.CompilerParams(needs_layout_passes=False),
    )
    def body(ks_hbm, vs_hbm, bnd_hbm, out_hbm,
             chA, chB, kwA, kwB, vwA, vwB, bnd_v,
             ksA, ksB, vsA, vsB, osA, osB):
        cid = lax.axis_index("c")
        sid = lax.axis_index("s")
        w = sid * _NC + cid
        pltpu.sync_copy(bnd_hbm, bnd_v)
        zvec = jnp.zeros((16,), jnp.float32)
        iota = lax.iota(jnp.int32, 16)

        def issue_win(t_base, kw, vw, ksem, vsem):
            base_c = jnp.minimum(t_base, clamp)
            pltpu.async_copy(ks_hbm.at[pl.ds(base_c, _WIN + 16)],
                             kw.at[pl.ds(0, _WIN + 16)], ksem)
            pltpu.async_copy(vs_hbm.at[pl.ds(base_c, _WIN + 16)],
                             vw.at[pl.ds(0, _WIN + 16)], vsem)

        def wait_win(kw, vw, ksem, vsem):
            pltpu.make_async_copy(ks_hbm.at[pl.ds(0, _WIN + 16)],
                                  kw.at[pl.ds(0, _WIN + 16)], ksem).wait()
            pltpu.make_async_copy(vs_hbm.at[pl.ds(0, _WIN + 16)],
                                  vw.at[pl.ds(0, _WIN + 16)], vsem).wait()

        def process_win(chunk_v, kw, vw, o0, nwords, lo16, hi, t):
            base = lo16 + t * _WIN
            shift = base - jnp.minimum(base, clamp)

            @plsc.parallel_loop(0, _GRP, unroll=4)
            def _(g):
                st = jnp.minimum(g * 16 + shift, _WIN + 16)
                k0 = kw[pl.ds(st, 16)]
                k1 = kw[pl.ds(st + 1, 16)]
                v = vw[pl.ds(st, 16)]
                local = k0 - o0
                pos = base + g * 16 + iota
                m = (((k0 != k1) | (pos == (N - 1)))
                     & (local >= 0) & (local < nwords) & (pos < hi))
                plsc.store_scatter(chunk_v, [local], v, mask=m)

        def do_chunk(kglob, o0, nwords, chunk_v, kws, vws, ksems, vsems):
            @plsc.parallel_loop(0, nwords // 16, unroll=8)
            def _(i):
                chunk_v[pl.ds(i * 16, 16)] = zvec

            bvec = bnd_v[pl.ds(kglob, 16)]
            lo = bvec[0]
            hi = bvec[1]
            lo16 = (lo // 16) * 16
            nwin = (hi - lo16 + (_WIN - 1)) // _WIN

            @pl.when(nwin > 0)
            def _():
                issue_win(lo16, kws[0], vws[0], ksems[0], vsems[0])

                def wpair(t, _c):
                    wait_win(kws[0], vws[0], ksems[0], vsems[0])

                    @pl.when(2 * t + 1 < nwin)
                    def _():
                        issue_win(lo16 + (2 * t + 1) * _WIN,
                                  kws[1], vws[1], ksems[1], vsems[1])

                    process_win(chunk_v, kws[0], vws[0], o0, nwords, lo16,
                                hi, 2 * t)

                    @pl.when(2 * t + 2 < nwin)
                    def _():
                        issue_win(lo16 + (2 * t + 2) * _WIN,
                                  kws[0], vws[0], ksems[0], vsems[0])

                    @pl.when(2 * t + 1 < nwin)
                    def _():
                        wait_win(kws[1], vws[1], ksems[1], vsems[1])
                        process_win(chunk_v, kws[1], vws[1], o0, nwords,
                                    lo16, hi, 2 * t + 1)

                    return 0

                lax.fori_loop(0, (nwin + 1) // 2, wpair, 0)

        def out_start(chunk_v, o0, nwords, osem):
            pltpu.async_copy(chunk_v.at[pl.ds(0, nwords)],
                             out_hbm.at[pl.ds(o0, nwords)], osem)

        def out_wait(chunk_v, nwords, osem):
            pltpu.make_async_copy(chunk_v.at[pl.ds(0, nwords)],
                                  out_hbm.at[pl.ds(0, nwords)], osem).wait()

        def cpair(i, _c):
            ci = 2 * i

            @pl.when(i > 0)
            def _():
                out_wait(chA, _CH, osA)

            do_chunk(w * npc + ci, w * R + ci * _CH, _CH, chA,
                     (kwA, kwB), (vwA, vwB), (ksA, ksB), (vsA, vsB))
            out_start(chA, w * R + ci * _CH, _CH, osA)

            @pl.when(i > 0)
            def _():
                out_wait(chB, _CH, osB)

            do_chunk(w * npc + ci + 1, w * R + (ci + 1) * _CH, _CH, chB,
                     (kwA, kwB), (vwA, vwB), (ksA, ksB), (vsA, vsB))
            out_start(chB, w * R + (ci + 1) * _CH, _CH, osB)
            return 0

        lax.fori_loop(0, n_half, cpair, 0)
        out_wait(chA, _CH, osA)
        do_chunk(w * npc + n_full, w * R + n_full * _CH, part, chA,
                 (kwA, kwB), (vwA, vwB), (ksA, ksB), (vsA, vsB))
        out_start(chA, w * R + n_full * _CH, part, osA)
        out_wait(chA, part, osA)
        out_wait(chB, _CH, osB)

    return body(ks, vs, bnd)


def kernel(inputs, unpool_mat, output_shape):
    B, Hp, Wp, C = inputs.shape
    Ho, Wo = 2 * Hp, 2 * Wp
    M = B * Ho * Wo * C
    N = B * Hp * Wp * C

    vals = jnp.reshape(inputs, (-1,))
    idx = jnp.reshape(unpool_mat, (-1,))
    if idx.dtype != jnp.int32:
        idx = idx.astype(jnp.int32)

    ks, vs = lax.sort((idx, vals), dimension=0, num_keys=1, is_stable=False)

    R = M // _NW
    n_full = (R // _CH) if (R % _CH) else (R // _CH - 1)
    if n_full % 2:
        n_full -= 1
    part = R - n_full * _CH
    assert part <= _CH
    npc = n_full + 1
    starts = []
    for w in range(_NW):
        for c in range(npc):
            starts.append(w * R + c * _CH)
    starts.append(M)
    q = jnp.asarray(starts, dtype=jnp.int32)
    bnd = jnp.searchsorted(ks, q).astype(jnp.int32)
    n_bnd = ((_NW * npc + 1 + 15) // 16) * 16
    bnd = jnp.concatenate(
        [bnd, jnp.full((n_bnd - bnd.shape[0],), N, jnp.int32)])

    out = _sc_scatter(ks, vs, bnd, M, n_full, part)
    return jnp.reshape(out, (B, Ho, Wo, C))
